# baseline (speedup 1.0000x reference)
import functools

import jax
import jax.numpy as jnp
from jax import lax
from jax.experimental import pallas as pl
from jax.experimental.pallas import tpu as pltpu
from jax.experimental.pallas import tpu_sc as plsc

N_NODES = 10000
N_EDGES = 320000
F = 128

NC = 2
NS = 16
NL = 16
NW = NC * NS
EPW = N_EDGES // NW
ECH = 80
NCHUNK = EPW // ECH
N_PAD = 10240
RPT = N_PAD // NS
ZR = 128


def _mm_body(x_ref, w_ref, b_ref, o_ref):
    o_ref[...] = (
        jnp.dot(x_ref[...], w_ref[...], preferred_element_type=jnp.float32)
        + b_ref[...]
    ).astype(jnp.bfloat16)


def _matmul(x, W, b):
    bm = 1000
    return pl.pallas_call(
        _mm_body,
        grid=(N_NODES // bm,),
        in_specs=[
            pl.BlockSpec((bm, F), lambda i: (i, 0)),
            pl.BlockSpec((F, F), lambda i: (0, 0)),
            pl.BlockSpec((1, F), lambda i: (0, 0)),
        ],
        out_specs=pl.BlockSpec((bm, F), lambda i: (i, 0)),
        out_shape=jax.ShapeDtypeStruct((N_NODES, F), jnp.bfloat16),
    )(x, W, b.reshape(1, F))


_MESH = plsc.VectorSubcoreMesh(
    core_axis_name="c", subcore_axis_name="s", num_cores=NC, num_subcores=NS
)


def _lane_bcast(v16, lane):
    return lax.gather(
        v16,
        jnp.full((NL, 1), lane, jnp.int32),
        lax.GatherDimensionNumbers(
            offset_dims=(), collapsed_slice_dims=(0,), start_index_map=(0,)
        ),
        slice_sizes=(1,),
        mode=lax.GatherScatterMode.PROMISE_IN_BOUNDS,
    )


def _maybe(cond, fn):
    if isinstance(cond, (bool, int)):
        if cond:
            fn()
    else:
        pl.when(cond)(fn)


def _spmm_body(h_hbm, src_hbm, dst_hbm, w_hbm, z_hbm, out_hbm, acc, *rest):
    D = 4
    sidx = rest[0:4]
    didx = rest[4:8]
    wst = rest[8:12]
    rbf = rest[12:16]
    rf32 = rest[16:18]
    gsem = rest[18:22]
    ssem = rest[22:26]
    isem = rest[26:30]
    dsem = rest[30:34]
    wsem = rest[34:38]

    c = lax.axis_index("c")
    s = lax.axis_index("s")
    wid = s * NC + c
    ebase = wid * EPW

    def stage_src(ch, q):
        pltpu.async_copy(src_hbm.at[pl.ds(ebase + ch * ECH, ECH)],
                         sidx[q], isem[q])

    def wait_src(q):
        pltpu.make_async_copy(src_hbm.at[pl.ds(0, ECH)], sidx[q],
                              isem[q]).wait()

    def stage_dst(ch, q):
        pltpu.async_copy(dst_hbm.at[pl.ds(ebase + ch * ECH, ECH)],
                         didx[q], dsem[q])

    def wait_dst(q):
        pltpu.make_async_copy(dst_hbm.at[pl.ds(0, ECH)], didx[q],
                              dsem[q]).wait()

    def stage_w(ch, q):
        pltpu.async_copy(w_hbm.at[pl.ds(ebase + ch * ECH, ECH)],
                         wst[q], wsem[q])

    def wait_w(q):
        pltpu.make_async_copy(w_hbm.at[pl.ds(0, ECH)], wst[q], wsem[q]).wait()

    def issue_g(q):
        pltpu.async_copy(h_hbm.at[sidx[q]], rbf[q], gsem[q])

    def wait_g(q):
        pltpu.make_async_copy(h_hbm.at[sidx[q]], rbf[q], gsem[q]).wait()

    def issue_s(q):
        pltpu.async_copy(rf32[q % 2], acc.at[didx[q]], ssem[q], add=True)

    def wait_s(q):
        pltpu.make_async_copy(rf32[q % 2], acc.at[didx[q]], ssem[q]).wait()

    def scale(q):
        def sub(rb, carry):
            w16 = wst[q][pl.ds(rb * NL, NL)]
            for r in range(NL):
                wb = _lane_bcast(w16, r)
                row = rb * NL + r
                for g in range(F // (2 * NL)):
                    v = rbf[q][row, pl.ds(g * NL, NL)]
                    lo = lax.bitcast_convert_type(v << 16, jnp.float32)
                    hi = lax.bitcast_convert_type(
                        v & jnp.int32(-65536), jnp.float32)
                    rf32[q % 2][row, pl.ds(g * 2 * NL, NL)] = lo * wb
                    rf32[q % 2][row, pl.ds(g * 2 * NL + NL, NL)] = hi * wb
            return carry

        lax.fori_loop(0, ECH // NL, sub, 0)

    for q0 in range(D):
        stage_src(q0, q0)
        stage_w(q0, q0)
    stage_dst(0, 0)
    stage_dst(1, 1)

    def zacc(i, carry):
        pltpu.sync_copy(z_hbm, acc.at[pl.ds(s * RPT + i * ZR, ZR)])
        return carry

    lax.fori_loop(0, RPT // ZR, zacc, 0)
    plsc.subcore_barrier()

    wait_src(0)
    issue_g(0)
    wait_src(1)
    issue_g(1)

    def chunk(ch, q, skip_s_wait=False):
        wait_g(q)
        if not skip_s_wait:
            wait_s((q + 2) % D)

        def ahead2():
            wait_src((q + 2) % D)
            issue_g((q + 2) % D)
            stage_dst(ch + 2, (q + 2) % D)

        _maybe(ch + 2 < NCHUNK, ahead2)
        _maybe(ch + 4 < NCHUNK, lambda: stage_src(ch + 4, q))
        wait_w(q)
        scale(q)
        _maybe(ch + 4 < NCHUNK, lambda: stage_w(ch + 4, q))
        wait_dst(q)
        issue_s(q)

    def quad(i, carry):
        base = 4 * i
        for u in range(D):
            chunk(base + u, u)
        return carry

    chunk(0, 0, skip_s_wait=True)
    chunk(1, 1, skip_s_wait=True)
    chunk(2, 2)
    chunk(3, 3)
    lax.fori_loop(1, (NCHUNK - 1) // D, quad, 0)
    chunk(NCHUNK - 1, (NCHUNK - 1) % 4)
    wait_s((NCHUNK - 2) % 4)
    wait_s((NCHUNK - 1) % 4)
    plsc.subcore_barrier()

    pltpu.sync_copy(
        acc.at[pl.ds(s * RPT, RPT)],
        out_hbm.at[c].at[pl.ds(s * RPT, RPT)],
    )


_spmm = functools.partial(
    pl.kernel,
    out_type=jax.ShapeDtypeStruct((NC, N_PAD, F), jnp.float32),
    mesh=_MESH,
    compiler_params=pltpu.CompilerParams(use_tc_tiling_on_sc=False),
    scratch_types=(
        [pltpu.VMEM_SHARED((N_PAD, F), jnp.float32)]
        + [pltpu.VMEM((ECH,), jnp.int32) for _ in range(4)]
        + [pltpu.VMEM((ECH,), jnp.int32) for _ in range(4)]
        + [pltpu.VMEM((ECH,), jnp.float32) for _ in range(4)]
        + [pltpu.VMEM((ECH, F // 2), jnp.int32) for _ in range(4)]
        + [pltpu.VMEM((ECH, F), jnp.float32) for _ in range(2)]
        + [pltpu.SemaphoreType.DMA for _ in range(20)]
    ),
)(_spmm_body)


def _add_body(p_ref, o_ref):
    o_ref[...] = p_ref[0] + p_ref[1]


def _pair_add(p):
    bm = 1024
    return pl.pallas_call(
        _add_body,
        grid=(N_PAD // bm,),
        in_specs=[pl.BlockSpec((NC, bm, F), lambda i: (0, i, 0))],
        out_specs=pl.BlockSpec((bm, F), lambda i: (i, 0)),
        out_shape=jax.ShapeDtypeStruct((N_PAD, F), jnp.float32),
    )(p)


_PERM = []
for _g in range(F // 32):
    for _k in range(16):
        _PERM.extend((_g * 32 + _k, _g * 32 + 16 + _k))
_INV = [0] * F
for _p, _cidx in enumerate(_PERM):
    _INV[_cidx] = _p


def kernel(x, edge_index, edge_weight, W, b):
    h = _matmul(x, W[:, jnp.array(_PERM)], b[jnp.array(_PERM)])
    h = lax.bitcast_convert_type(h.reshape(N_NODES, F // 2, 2), jnp.int32)
    dst = edge_index[0].astype(jnp.int32)
    src = edge_index[1].astype(jnp.int32)
    zeros = jnp.zeros((ZR, F), jnp.float32)
    partial = _spmm(h, src, dst, edge_weight, zeros)
    return _pair_add(partial)[:N_NODES]


# speedup vs baseline: 1.9662x; 1.9662x over previous
import functools

import jax
import jax.numpy as jnp
from jax import lax
from jax.experimental import pallas as pl
from jax.experimental.pallas import tpu as pltpu
from jax.experimental.pallas import tpu_sc as plsc

N_NODES = 10000
N_EDGES = 320000
F = 128

NC = 2
NS = 16
NL = 16
NW = NC * NS
EPW = N_EDGES // NW
ECH = 80
NCHUNK = EPW // ECH
N_PAD = 10240
RPT = N_PAD // NS
ZR = 128


def _mm_body(x_ref, w_ref, b_ref, o_ref):
    o_ref[...] = (
        jnp.dot(x_ref[...], w_ref[...], preferred_element_type=jnp.float32)
        + b_ref[...]
    )


def _matmul(x, W, b):
    bm = 1000
    return pl.pallas_call(
        _mm_body,
        grid=(N_NODES // bm,),
        in_specs=[
            pl.BlockSpec((bm, F), lambda i: (i, 0)),
            pl.BlockSpec((F, F), lambda i: (0, 0)),
            pl.BlockSpec((1, F), lambda i: (0, 0)),
        ],
        out_specs=pl.BlockSpec((bm, F), lambda i: (i, 0)),
        out_shape=jax.ShapeDtypeStruct((N_NODES, F), jnp.float32),
    )(x, W, b.reshape(1, F))


_MESH = plsc.VectorSubcoreMesh(
    core_axis_name="c", subcore_axis_name="s", num_cores=NC, num_subcores=NS
)


def _lane_bcast(v16, lane):
    return lax.gather(
        v16,
        jnp.full((NL, 1), lane, jnp.int32),
        lax.GatherDimensionNumbers(
            offset_dims=(), collapsed_slice_dims=(0,), start_index_map=(0,)
        ),
        slice_sizes=(1,),
        mode=lax.GatherScatterMode.PROMISE_IN_BOUNDS,
    )


def _maybe(cond, fn):
    if isinstance(cond, (bool, int)):
        if cond:
            fn()
    else:
        pl.when(cond)(fn)


def _spmm_body(h_hbm, src_hbm, dst_hbm, w_hbm, z_hbm, out_hbm, acc, *rest):
    D = 4
    sidx = rest[0:4]
    didx = rest[4:8]
    wst = rest[8:12]
    rows = rest[12:16]
    gsem = rest[16:20]
    ssem = rest[20:24]
    isem = rest[24:28]
    dsem = rest[28:32]
    wsem = rest[32:36]

    c = lax.axis_index("c")
    s = lax.axis_index("s")
    wid = s * NC + c
    ebase = wid * EPW

    def stage_src(ch, q):
        pltpu.async_copy(src_hbm.at[pl.ds(ebase + ch * ECH, ECH)],
                         sidx[q], isem[q])

    def wait_src(q):
        pltpu.make_async_copy(src_hbm.at[pl.ds(0, ECH)], sidx[q],
                              isem[q]).wait()

    def stage_dst(ch, q):
        pltpu.async_copy(dst_hbm.at[pl.ds(ebase + ch * ECH, ECH)],
                         didx[q], dsem[q])

    def wait_dst(q):
        pltpu.make_async_copy(dst_hbm.at[pl.ds(0, ECH)], didx[q],
                              dsem[q]).wait()

    def stage_w(ch, q):
        pltpu.async_copy(w_hbm.at[pl.ds(ebase + ch * ECH, ECH)],
                         wst[q], wsem[q])

    def wait_w(q):
        pltpu.make_async_copy(w_hbm.at[pl.ds(0, ECH)], wst[q], wsem[q]).wait()

    def issue_g(q):
        pltpu.async_copy(h_hbm.at[sidx[q]], rows[q], gsem[q])

    def wait_g(q):
        pltpu.make_async_copy(h_hbm.at[sidx[q]], rows[q], gsem[q]).wait()

    def issue_s(q):
        pltpu.async_copy(rows[q], acc.at[didx[q]], ssem[q], add=True)

    def wait_s(q):
        pltpu.make_async_copy(rows[q], acc.at[didx[q]], ssem[q]).wait()

    def scale(q):
        def sub(rb, carry):
            w16 = wst[q][pl.ds(rb * NL, NL)]
            for r in range(NL):
                wb = _lane_bcast(w16, r)
                row = rb * NL + r
                for j in range(F // NL):
                    sl = pl.ds(j * NL, NL)
                    rows[q][row, sl] = rows[q][row, sl] * wb
            return carry

        lax.fori_loop(0, ECH // NL, sub, 0)

    for q0 in range(D):
        stage_src(q0, q0)
        stage_w(q0, q0)
    stage_dst(0, 0)
    stage_dst(1, 1)

    def zacc(i, carry):
        pltpu.sync_copy(z_hbm, acc.at[pl.ds(s * RPT + i * ZR, ZR)])
        return carry

    lax.fori_loop(0, RPT // ZR, zacc, 0)
    plsc.subcore_barrier()

    wait_src(0)
    issue_g(0)
    wait_src(1)
    issue_g(1)

    def chunk(ch, q, skip_s_wait=False):
        wait_g(q)
        if not skip_s_wait:
            wait_s((q + 2) % D)

        def ahead2():
            wait_src((q + 2) % D)
            issue_g((q + 2) % D)
            stage_dst(ch + 2, (q + 2) % D)

        _maybe(ch + 2 < NCHUNK, ahead2)
        _maybe(ch + 4 < NCHUNK, lambda: stage_src(ch + 4, q))
        wait_w(q)
        scale(q)
        _maybe(ch + 4 < NCHUNK, lambda: stage_w(ch + 4, q))
        wait_dst(q)
        issue_s(q)

    def quad(i, carry):
        base = 4 * i
        for u in range(D):
            chunk(base + u, u)
        return carry

    chunk(0, 0, skip_s_wait=True)
    chunk(1, 1, skip_s_wait=True)
    chunk(2, 2)
    chunk(3, 3)
    lax.fori_loop(1, (NCHUNK - 1) // D, quad, 0)
    chunk(NCHUNK - 1, (NCHUNK - 1) % 4)
    wait_s((NCHUNK - 2) % 4)
    wait_s((NCHUNK - 1) % 4)
    plsc.subcore_barrier()

    pltpu.sync_copy(
        acc.at[pl.ds(s * RPT, RPT)],
        out_hbm.at[c].at[pl.ds(s * RPT, RPT)],
    )


_spmm = functools.partial(
    pl.kernel,
    out_type=jax.ShapeDtypeStruct((NC, N_PAD, F), jnp.float32),
    mesh=_MESH,
    scratch_types=(
        [pltpu.VMEM_SHARED((N_PAD, F), jnp.float32)]
        + [pltpu.VMEM((ECH,), jnp.int32) for _ in range(4)]
        + [pltpu.VMEM((ECH,), jnp.int32) for _ in range(4)]
        + [pltpu.VMEM((ECH,), jnp.float32) for _ in range(4)]
        + [pltpu.VMEM((ECH, F), jnp.float32) for _ in range(4)]
        + [pltpu.SemaphoreType.DMA for _ in range(20)]
    ),
)(_spmm_body)


def _add_body(p_ref, o_ref):
    o_ref[...] = p_ref[0] + p_ref[1]


def _pair_add(p):
    bm = 1024
    return pl.pallas_call(
        _add_body,
        grid=(N_PAD // bm,),
        in_specs=[pl.BlockSpec((NC, bm, F), lambda i: (0, i, 0))],
        out_specs=pl.BlockSpec((bm, F), lambda i: (i, 0)),
        out_shape=jax.ShapeDtypeStruct((N_PAD, F), jnp.float32),
    )(p)


def kernel(x, edge_index, edge_weight, W, b):
    h = _matmul(x, W, b)
    dst = edge_index[0].astype(jnp.int32)
    src = edge_index[1].astype(jnp.int32)
    zeros = jnp.zeros((ZR, F), jnp.float32)
    partial = _spmm(h, src, dst, edge_weight, zeros)
    return _pair_add(partial)[:N_NODES]
